# baseline (speedup 1.0000x reference)
import functools

import jax
import jax.numpy as jnp
from jax import lax
from jax.experimental import pallas as pl
from jax.experimental.pallas import tpu as pltpu

_TAU = 0.1
_UNIFORM_WEIGHT = 0.1
_TOPK = 5
_TILE = 512
_LOG2E = 1.4426950408889634


def _precompute_kernel(z_ref, s_ref, zn_ref, sqs_row_ref, kap_col_ref,
                       kap_row_ref):
    S = s_ref[...]
    sqs = jnp.sum(S * S, axis=1, keepdims=True)
    sqs_row_ref[...] = sqs.T + 512.0
    Z = z_ref[...]
    nsq = jnp.sum(Z * Z, axis=1, keepdims=True)
    r = 1.0 / jnp.maximum(jnp.sqrt(nsq), 1e-12)
    zn_ref[...] = (Z * r).astype(jnp.bfloat16)
    kap = jnp.exp(-2.0 * (nsq * r * r))
    kap_col_ref[...] = kap
    kap_row_ref[...] = kap.T


def _loss_kernel(zn_ref, s_ref, sqs_row_ref, kap_col_ref, kap_row_ref,
                 choice_ref, info_ref, unif_ref, *, k, tile):
    i = pl.program_id(0)
    B = s_ref.shape[0]
    row0 = i * tile

    sm2 = s_ref[pl.ds(row0, tile), :] * (-2.0)
    G2 = lax.dot_general(sm2, s_ref[...], (((1,), (1,)), ((), ())),
                         preferred_element_type=jnp.float32)
    v = G2 + sqs_row_ref[...]
    col = lax.broadcasted_iota(jnp.int32, (tile, B), 1)
    row = lax.broadcasted_iota(jnp.int32, (tile, B), 0) + row0
    inf = jnp.float32(jnp.inf)
    v = jnp.where(col == row, jnp.float32(3.0e38), v)
    ki = (lax.bitcast_convert_type(v, jnp.int32) & jnp.int32(~0xFFF)) | col
    key = lax.bitcast_convert_type(ki, jnp.float32)

    choice = choice_ref[0]
    pos = jnp.zeros((tile, 1), jnp.int32)
    for rnd in range(k):
        mkey = jnp.min(key, axis=1, keepdims=True)
        idx = lax.bitcast_convert_type(mkey, jnp.int32) & jnp.int32(0xFFF)
        pos = jnp.where(choice == rnd, idx, pos)
        if rnd + 1 < k:
            key = jnp.where(key == mkey, inf, key)

    zn_i = zn_ref[pl.ds(row0, tile), :]
    sim = lax.dot_general(zn_i, zn_ref[...], (((1,), (1,)), ((), ())),
                          preferred_element_type=jnp.float32)
    t = jnp.exp2(sim * jnp.float32(2.0 * _LOG2E))
    t2 = t * t
    t4 = t2 * t2
    e = t4 * t
    denom = jnp.sum(e, axis=1, keepdims=True)
    numer = jnp.sum(jnp.where(col == pos, e, 0.0), axis=1, keepdims=True)
    info = jnp.sum(-jnp.log(numer / (denom + 1e-8)), keepdims=True)

    w = jnp.sum(t2 * kap_row_ref[...], axis=1, keepdims=True)
    usum = jnp.sum(w * kap_col_ref[pl.ds(row0, tile), :], keepdims=True)

    info_ref[...] = info.reshape(1, 1, 1)
    unif_ref[...] = usum.reshape(1, 1, 1)


def kernel(z, binding_scores):
    B, D = z.shape
    F = binding_scores.shape[1]
    k = min(_TOPK, B - 1)
    tile = _TILE if B % _TILE == 0 else B
    nsteps = B // tile
    choice = jax.random.randint(jax.random.key(12345), (B,), 0, k)
    choice3 = choice.astype(jnp.int32).reshape(nsteps, tile, 1)

    zn, sqs_row, kap_col, kap_row = pl.pallas_call(
        _precompute_kernel,
        in_specs=[pl.BlockSpec((B, D), lambda: (0, 0)),
                  pl.BlockSpec((B, F), lambda: (0, 0))],
        out_specs=[pl.BlockSpec((B, D), lambda: (0, 0)),
                   pl.BlockSpec((1, B), lambda: (0, 0)),
                   pl.BlockSpec((B, 1), lambda: (0, 0)),
                   pl.BlockSpec((1, B), lambda: (0, 0))],
        out_shape=[jax.ShapeDtypeStruct((B, D), jnp.bfloat16),
                   jax.ShapeDtypeStruct((1, B), jnp.float32),
                   jax.ShapeDtypeStruct((B, 1), jnp.float32),
                   jax.ShapeDtypeStruct((1, B), jnp.float32)],
    )(z, binding_scores)

    body = functools.partial(_loss_kernel, k=k, tile=tile)
    info_parts, unif_parts = pl.pallas_call(
        body,
        grid=(nsteps,),
        in_specs=[
            pl.BlockSpec((B, D), lambda i: (0, 0)),
            pl.BlockSpec((B, F), lambda i: (0, 0)),
            pl.BlockSpec((1, B), lambda i: (0, 0)),
            pl.BlockSpec((B, 1), lambda i: (0, 0)),
            pl.BlockSpec((1, B), lambda i: (0, 0)),
            pl.BlockSpec((1, tile, 1), lambda i: (i, 0, 0)),
        ],
        out_specs=[pl.BlockSpec((1, 1, 1), lambda i: (i, 0, 0)),
                   pl.BlockSpec((1, 1, 1), lambda i: (i, 0, 0))],
        out_shape=[jax.ShapeDtypeStruct((nsteps, 1, 1), jnp.float32),
                   jax.ShapeDtypeStruct((nsteps, 1, 1), jnp.float32)],
        compiler_params=pltpu.CompilerParams(
            dimension_semantics=("parallel",)),
    )(zn, binding_scores, sqs_row, kap_col, kap_row, choice3)

    L_info = jnp.sum(info_parts) / B
    L_unif = jnp.log(jnp.sum(unif_parts) / (B * B) + 1e-8)
    return L_info + _UNIFORM_WEIGHT * L_unif


# speedup vs baseline: 1.0687x; 1.0687x over previous
import functools

import jax
import jax.numpy as jnp
from jax import lax
from jax.experimental import pallas as pl
from jax.experimental.pallas import tpu as pltpu

_TAU = 0.1
_UNIFORM_WEIGHT = 0.1
_TOPK = 5
_TILE = 512
_LOG2E = 1.4426950408889634


def _loss_kernel(z_ref, s_ref, choice_ref, info_ref, unif_ref,
                 zn_ref, sb_ref, sqs_row_ref, kap_col_ref, kap_row_ref,
                 *, k, tile):
    i = pl.program_id(0)
    B = z_ref.shape[0]
    row0 = i * tile

    @pl.when(i == 0)
    def _():
        S = s_ref[...]
        sqs = jnp.sum(S * S, axis=1, keepdims=True)
        sqs_row_ref[...] = sqs.T + 512.0
        sb_ref[...] = S.astype(jnp.bfloat16)
        Z = z_ref[...]
        nsq = jnp.sum(Z * Z, axis=1, keepdims=True)
        r = 1.0 / jnp.maximum(jnp.sqrt(nsq), 1e-12)
        zn_ref[...] = (Z * r).astype(jnp.bfloat16)
        kap = jnp.exp(-2.0 * (nsq * r * r))
        kap_col_ref[...] = kap
        kap_row_ref[...] = kap.T
        info_ref[...] = jnp.zeros((1, 1), jnp.float32)
        unif_ref[...] = jnp.zeros((1, 1), jnp.float32)

    sm2 = sb_ref[pl.ds(row0, tile), :] * jnp.bfloat16(-2.0)
    G2 = lax.dot_general(sm2, sb_ref[...], (((1,), (1,)), ((), ())),
                         preferred_element_type=jnp.float32)
    zn_i = zn_ref[pl.ds(row0, tile), :]
    sim = lax.dot_general(zn_i, zn_ref[...], (((1,), (1,)), ((), ())),
                          preferred_element_type=jnp.float32)

    v = G2 + sqs_row_ref[...]
    col = lax.broadcasted_iota(jnp.int32, (tile, B), 1)
    row_l = lax.broadcasted_iota(jnp.int32, (tile, 1), 0) + row0
    inf = jnp.float32(jnp.inf)
    v = jnp.where(col == row_l, jnp.float32(3.0e38), v)
    ki = (lax.bitcast_convert_type(v, jnp.int32) & jnp.int32(~0xFFF)) | col
    key = lax.bitcast_convert_type(ki, jnp.float32)

    choice = choice_ref[0]
    pos = jnp.zeros((tile, 1), jnp.int32)
    for rnd in range(k):
        mkey = jnp.min(key, axis=1, keepdims=True)
        idx = lax.bitcast_convert_type(mkey, jnp.int32) & jnp.int32(0xFFF)
        pos = jnp.where(choice == rnd, idx, pos)
        if rnd + 1 < k:
            key = jnp.where(key == mkey, inf, key)

    t = jnp.exp2(sim * jnp.float32(2.0 * _LOG2E))
    t2 = t * t
    t4 = t2 * t2
    e = t4 * t
    denom = jnp.sum(e, axis=1, keepdims=True)
    numer = jnp.sum(jnp.where(col == pos, e, 0.0), axis=1, keepdims=True)
    info = jnp.sum(-jnp.log(numer / (denom + 1e-8)), keepdims=True)

    w = jnp.sum(t2 * kap_row_ref[...], axis=1, keepdims=True)
    usum = jnp.sum(w * kap_col_ref[pl.ds(row0, tile), :], keepdims=True)

    info_ref[...] += info.reshape(1, 1)
    unif_ref[...] += usum.reshape(1, 1)


def kernel(z, binding_scores):
    B, D = z.shape
    F = binding_scores.shape[1]
    k = min(_TOPK, B - 1)
    tile = _TILE if B % _TILE == 0 else B
    nsteps = B // tile
    choice = jax.random.randint(jax.random.key(12345), (B,), 0, k)
    choice3 = choice.astype(jnp.int32).reshape(nsteps, tile, 1)
    body = functools.partial(_loss_kernel, k=k, tile=tile)
    info_sum, unif_sum = pl.pallas_call(
        body,
        grid=(nsteps,),
        in_specs=[
            pl.BlockSpec((B, D), lambda i: (0, 0)),
            pl.BlockSpec((B, F), lambda i: (0, 0)),
            pl.BlockSpec((1, tile, 1), lambda i: (i, 0, 0)),
        ],
        out_specs=[pl.BlockSpec((1, 1), lambda i: (0, 0)),
                   pl.BlockSpec((1, 1), lambda i: (0, 0))],
        out_shape=[jax.ShapeDtypeStruct((1, 1), jnp.float32),
                   jax.ShapeDtypeStruct((1, 1), jnp.float32)],
        scratch_shapes=[
            pltpu.VMEM((B, D), jnp.bfloat16),
            pltpu.VMEM((B, F), jnp.bfloat16),
            pltpu.VMEM((1, B), jnp.float32),
            pltpu.VMEM((B, 1), jnp.float32),
            pltpu.VMEM((1, B), jnp.float32),
        ],
    )(z, binding_scores, choice3)
    L_info = info_sum[0, 0] / B
    L_unif = jnp.log(unif_sum[0, 0] / (B * B) + 1e-8)
    return L_info + _UNIFORM_WEIGHT * L_unif


# speedup vs baseline: 1.0699x; 1.0011x over previous
import functools

import jax
import jax.numpy as jnp
from jax import lax
from jax.experimental import pallas as pl
from jax.experimental.pallas import tpu as pltpu

_TAU = 0.1
_UNIFORM_WEIGHT = 0.1
_TOPK = 5
_TILE = 512
_LOG2E = 1.4426950408889634


def _loss_kernel(z_ref, s_ref, choice_ref, info_ref, unif_ref,
                 zn_ref, sb_ref, sqs_row_ref, *, k, tile):
    i = pl.program_id(0)
    B = z_ref.shape[0]
    row0 = i * tile

    @pl.when(i == 0)
    def _():
        S = s_ref[...]
        sqs = jnp.sum(S * S, axis=1, keepdims=True)
        sqs_row_ref[...] = sqs.T + 512.0
        sb_ref[...] = S.astype(jnp.bfloat16)
        Z = z_ref[...]
        nsq = jnp.sum(Z * Z, axis=1, keepdims=True)
        r = 1.0 / jnp.maximum(jnp.sqrt(nsq), 1e-12)
        zn_ref[...] = (Z * r).astype(jnp.bfloat16)
        info_ref[...] = jnp.zeros((1, 1), jnp.float32)
        unif_ref[...] = jnp.zeros((1, 1), jnp.float32)

    sm2 = sb_ref[pl.ds(row0, tile), :] * jnp.bfloat16(-2.0)
    G2 = lax.dot_general(sm2, sb_ref[...], (((1,), (1,)), ((), ())),
                         preferred_element_type=jnp.float32)
    zn_i = zn_ref[pl.ds(row0, tile), :]
    sim = lax.dot_general(zn_i, zn_ref[...], (((1,), (1,)), ((), ())),
                          preferred_element_type=jnp.float32)

    v = G2 + sqs_row_ref[...]
    col = lax.broadcasted_iota(jnp.int32, (tile, B), 1)
    row_l = lax.broadcasted_iota(jnp.int32, (tile, 1), 0) + row0
    inf = jnp.float32(jnp.inf)
    v = jnp.where(col == row_l, jnp.float32(3.0e38), v)
    ki = (lax.bitcast_convert_type(v, jnp.int32) & jnp.int32(~0xFFF)) | col
    key = lax.bitcast_convert_type(ki, jnp.float32)

    choice = choice_ref[0]
    pos = jnp.zeros((tile, 1), jnp.int32)
    for rnd in range(k):
        mkey = jnp.min(key, axis=1, keepdims=True)
        idx = lax.bitcast_convert_type(mkey, jnp.int32) & jnp.int32(0xFFF)
        pos = jnp.where(choice == rnd, idx, pos)
        if rnd + 1 < k:
            key = jnp.where(key == mkey, inf, key)

    e = jnp.exp2(sim * jnp.float32(10.0 * _LOG2E))
    u = jnp.exp2(sim * jnp.float32(4.0 * _LOG2E))
    denom = jnp.sum(e, axis=1, keepdims=True)
    numer = jnp.sum(jnp.where(col == pos, e, 0.0), axis=1, keepdims=True)
    info = jnp.sum(-jnp.log(numer / (denom + 1e-8)), keepdims=True)
    usum = jnp.sum(u, keepdims=True)

    info_ref[...] += info.reshape(1, 1)
    unif_ref[...] += usum.reshape(1, 1)


def kernel(z, binding_scores):
    B, D = z.shape
    F = binding_scores.shape[1]
    k = min(_TOPK, B - 1)
    tile = _TILE if B % _TILE == 0 else B
    nsteps = B // tile
    choice = jax.random.randint(jax.random.key(12345), (B,), 0, k)
    choice3 = choice.astype(jnp.int32).reshape(nsteps, tile, 1)
    body = functools.partial(_loss_kernel, k=k, tile=tile)
    info_sum, unif_sum = pl.pallas_call(
        body,
        grid=(nsteps,),
        in_specs=[
            pl.BlockSpec((B, D), lambda i: (0, 0)),
            pl.BlockSpec((B, F), lambda i: (0, 0)),
            pl.BlockSpec((1, tile, 1), lambda i: (i, 0, 0)),
        ],
        out_specs=[pl.BlockSpec((1, 1), lambda i: (0, 0)),
                   pl.BlockSpec((1, 1), lambda i: (0, 0))],
        out_shape=[jax.ShapeDtypeStruct((1, 1), jnp.float32),
                   jax.ShapeDtypeStruct((1, 1), jnp.float32)],
        scratch_shapes=[
            pltpu.VMEM((B, D), jnp.bfloat16),
            pltpu.VMEM((B, F), jnp.bfloat16),
            pltpu.VMEM((1, B), jnp.float32),
        ],
    )(z, binding_scores, choice3)
    L_info = info_sum[0, 0] / B
    L_unif = jnp.log(unif_sum[0, 0] * jnp.exp(-4.0) / (B * B) + 1e-8)
    return L_info + _UNIFORM_WEIGHT * L_unif


# speedup vs baseline: 1.0727x; 1.0026x over previous
import functools

import jax
import jax.numpy as jnp
from jax import lax
from jax.experimental import pallas as pl
from jax.experimental.pallas import tpu as pltpu

_TAU = 0.1
_UNIFORM_WEIGHT = 0.1
_TOPK = 5
_TILE = 512
_LOG2E = 1.4426950408889634


def _loss_kernel(z_ref, s_ref, choice_ref, info_ref, unif_ref,
                 zn_ref, sb_ref, sqs_row_ref, *, k, tile):
    i = pl.program_id(0)
    B = z_ref.shape[0]
    row0 = i * tile

    @pl.when(i == 0)
    def _():
        S = s_ref[...]
        sqs = jnp.sum(S * S, axis=1, keepdims=True)
        sqs_row_ref[...] = sqs.T + 512.0
        sb_ref[...] = S.astype(jnp.bfloat16)
        Z = z_ref[...]
        nsq = jnp.sum(Z * Z, axis=1, keepdims=True)
        r = 1.0 / jnp.maximum(jnp.sqrt(nsq), 1e-12)
        zn_ref[...] = (Z * r).astype(jnp.bfloat16)
        info_ref[...] = jnp.zeros((1, 1), jnp.float32)
        unif_ref[...] = jnp.zeros((1, 1), jnp.float32)

    sm2 = sb_ref[pl.ds(row0, tile), :] * jnp.bfloat16(-2.0)
    G2 = lax.dot_general(sm2, sb_ref[...], (((1,), (1,)), ((), ())),
                         preferred_element_type=jnp.float32)
    zn_i = zn_ref[pl.ds(row0, tile), :]
    sim = lax.dot_general(zn_i, zn_ref[...], (((1,), (1,)), ((), ())),
                          preferred_element_type=jnp.float32)

    v = G2 + sqs_row_ref[...]
    col = lax.broadcasted_iota(jnp.int32, (tile, B), 1)
    row_l = lax.broadcasted_iota(jnp.int32, (tile, 1), 0) + row0
    inf = jnp.float32(jnp.inf)
    v = jnp.where(col == row_l, jnp.float32(3.0e38), v)
    ki = (lax.bitcast_convert_type(v, jnp.int32) & jnp.int32(~0xFFF)) | col
    key = lax.bitcast_convert_type(ki, jnp.float32)

    choice = choice_ref[0]
    pos = jnp.zeros((tile, 1), jnp.int32)
    mkey = jnp.min(key, axis=1, keepdims=True)
    for rnd in range(k):
        idx = lax.bitcast_convert_type(mkey, jnp.int32) & jnp.int32(0xFFF)
        pos = jnp.where(choice == rnd, idx, pos)
        if rnd + 1 < k:
            mkey = jnp.min(jnp.where(key > mkey, key, inf),
                           axis=1, keepdims=True)

    e = jnp.exp2(sim * jnp.float32(10.0 * _LOG2E))
    u = jnp.exp2(sim * jnp.float32(4.0 * _LOG2E))
    denom = jnp.sum(e, axis=1, keepdims=True)
    numer = jnp.sum(jnp.where(col == pos, e, 0.0), axis=1, keepdims=True)
    info = jnp.sum(-jnp.log(numer / (denom + 1e-8)), keepdims=True)
    usum = jnp.sum(u, keepdims=True)

    info_ref[...] += info.reshape(1, 1)
    unif_ref[...] += usum.reshape(1, 1)


def kernel(z, binding_scores):
    B, D = z.shape
    F = binding_scores.shape[1]
    k = min(_TOPK, B - 1)
    tile = _TILE if B % _TILE == 0 else B
    nsteps = B // tile
    choice = jax.random.randint(jax.random.key(12345), (B,), 0, k)
    choice3 = choice.astype(jnp.int32).reshape(nsteps, tile, 1)
    body = functools.partial(_loss_kernel, k=k, tile=tile)
    info_sum, unif_sum = pl.pallas_call(
        body,
        grid=(nsteps,),
        in_specs=[
            pl.BlockSpec((B, D), lambda i: (0, 0)),
            pl.BlockSpec((B, F), lambda i: (0, 0)),
            pl.BlockSpec((1, tile, 1), lambda i: (i, 0, 0)),
        ],
        out_specs=[pl.BlockSpec((1, 1), lambda i: (0, 0)),
                   pl.BlockSpec((1, 1), lambda i: (0, 0))],
        out_shape=[jax.ShapeDtypeStruct((1, 1), jnp.float32),
                   jax.ShapeDtypeStruct((1, 1), jnp.float32)],
        scratch_shapes=[
            pltpu.VMEM((B, D), jnp.bfloat16),
            pltpu.VMEM((B, F), jnp.bfloat16),
            pltpu.VMEM((1, B), jnp.float32),
        ],
    )(z, binding_scores, choice3)
    L_info = info_sum[0, 0] / B
    L_unif = jnp.log(unif_sum[0, 0] * jnp.exp(-4.0) / (B * B) + 1e-8)
    return L_info + _UNIFORM_WEIGHT * L_unif
